# baseline (speedup 1.0000x reference)
import functools

import jax
import jax.numpy as jnp
from jax import lax
from jax.experimental import pallas as pl
from jax.experimental.pallas import tpu as pltpu
from jax.experimental.pallas import tpu_sc as plsc

_N = 10000
_E = 320000
_D = 128
_G = 8

_NSUB = 32
_ROWS = 320
_NPAD = _NSUB * _ROWS
_CHUNK = 2560
_NCHUNK = _E // _CHUNK
_NPAIR = _CHUNK // 32
_FB = 128
_NEG = float("-inf")


_HI = lax.Precision.HIGHEST


def _graph_norm_body(x, bt, w, b, ms):
    oh = (bt[:, None] == lax.broadcasted_iota(jnp.int32, (1, _G), 1))
    oh = oh.astype(jnp.float32)
    cnt = jnp.maximum(jnp.sum(oh, axis=0), 1.0)
    sums = lax.dot_general(oh, x, (((0,), (0,)), ((), ())), precision=_HI,
                           preferred_element_type=jnp.float32)
    mean = sums / cnt[:, None]
    meanb = jnp.dot(oh, mean, precision=_HI,
                    preferred_element_type=jnp.float32)
    out0 = x - ms * meanb
    var = lax.dot_general(oh, out0 * out0, (((0,), (0,)), ((), ())),
                          precision=_HI,
                          preferred_element_type=jnp.float32) / cnt[:, None]
    istd = lax.rsqrt(var + 1e-5)
    istdb = jnp.dot(oh, istd, precision=_HI,
                    preferred_element_type=jnp.float32)
    return w * (out0 * istdb) + b


def _gn1_kernel(x_ref, bt_ref, w_ref, b_ref, ms_ref, o_ref):
    o_ref[...] = _graph_norm_body(x_ref[...], bt_ref[...], w_ref[...],
                                  b_ref[...], ms_ref[...])


def _tc_gn1(x, batch, w, b, ms):
    return pl.pallas_call(
        _gn1_kernel,
        out_shape=jax.ShapeDtypeStruct((_N, _D), jnp.float32),
    )(x, batch, w, b, ms)


def _tail_kernel(x_ref, h_ref, agg_ref, Wl_ref, bl_ref, Wr_ref, bt_ref,
                 w_ref, b_ref, ms_ref, o_ref):
    agg = agg_ref[...]
    agg = jnp.where(agg == _NEG, 0.0, agg)
    y = (lax.dot_general(agg, Wl_ref[...], (((1,), (1,)), ((), ())),
                         precision=_HI, preferred_element_type=jnp.float32)
         + bl_ref[...]
         + lax.dot_general(h_ref[...], Wr_ref[...], (((1,), (1,)), ((), ())),
                           precision=_HI, preferred_element_type=jnp.float32))
    r = jnp.maximum(x_ref[...] + y, 0.0)
    o_ref[...] = _graph_norm_body(r, bt_ref[...], w_ref[...], b_ref[...],
                                  ms_ref[...])


def _tc_tail(x, h, agg, Wl, bl, Wr, batch, w2, b2, ms2):
    return pl.pallas_call(
        _tail_kernel,
        out_shape=jax.ShapeDtypeStruct((_N, _D), jnp.float32),
    )(x, h, agg, Wl, bl, Wr, batch, w2, b2, ms2)


def _sc_scatter_max(h, src, dst):
    mesh = plsc.VectorSubcoreMesh(core_axis_name="c", subcore_axis_name="s")

    @functools.partial(
        pl.kernel,
        out_type=jax.ShapeDtypeStruct((_NPAD, _D), jnp.float32),
        mesh=mesh,
        compiler_params=pltpu.CompilerParams(needs_layout_passes=False),
        scratch_types=[
            pltpu.VMEM((_ROWS, _D), jnp.float32),
            pltpu.VMEM((_CHUNK,), jnp.int32),
            pltpu.VMEM((_CHUNK,), jnp.int32),
            pltpu.VMEM((2 * _FB,), jnp.int32),
            pltpu.VMEM((2 * _FB + 16,), jnp.int32),
            pltpu.VMEM((2 * _FB, _D), jnp.float32),
            pltpu.SemaphoreType.DMA,
        ],
    )
    def body(h_hbm, src_hbm, dst_hbm, out_hbm, agg, dbuf, sbuf, csrc, cdl,
             rows, sem):
        wid = lax.axis_index("s") * 2 + lax.axis_index("c")
        lo = wid * _ROWS

        neg = jnp.full((16,), _NEG, jnp.float32)
        zero16 = jnp.zeros((16,), jnp.int32)

        def init_row(r, carry):
            for c in range(_D // 16):
                agg[r, pl.ds(c * 16, 16)] = neg
            return carry
        lax.fori_loop(0, _ROWS, init_row, 0)
        for g in range(2 * _FB // 16):
            csrc[pl.ds(g * 16, 16)] = zero16

        def start_gather(sl):
            pltpu.async_copy(h_hbm.at[csrc.at[pl.ds(sl * _FB, _FB)]],
                             rows.at[pl.ds(sl * _FB, _FB)], sem)

        def wait_gather(sl):
            pltpu.make_async_copy(h_hbm.at[csrc.at[pl.ds(sl * _FB, _FB)]],
                                  rows.at[pl.ds(sl * _FB, _FB)], sem).wait()

        def apply_slot(sl, n):
            base = sl * _FB

            def apply(j, carry):
                dl = cdl[pl.ds(base + j, 16)][0]
                for c in range(_D // 16):
                    slc = pl.ds(c * 16, 16)
                    agg[dl, slc] = jnp.maximum(agg[dl, slc],
                                               rows[base + j, slc])
                return carry
            lax.fori_loop(0, n, apply, 0)

        def chunk_body(k, st):
            pltpu.sync_copy(dst_hbm.at[pl.ds(k * _CHUNK, _CHUNK)], dbuf)
            pltpu.sync_copy(src_hbm.at[pl.ds(k * _CHUNK, _CHUNK)], sbuf)

            def inner_cond(t):
                i, p, slot, pend, pptr = t
                return (i < _CHUNK // 16) & (p <= _FB - 16)

            def inner_body(t):
                i, p, slot, pend, pptr = t
                d = dbuf[pl.ds(i * 16, 16)]
                s = sbuf[pl.ds(i * 16, 16)]
                dl = d - lo
                m = (dl >= 0) & (dl < _ROWS)
                plsc.store_compressed(csrc.at[pl.ds(slot * _FB + p, 16)], s,
                                      mask=m)
                plsc.store_compressed(cdl.at[pl.ds(slot * _FB + p, 16)], dl,
                                      mask=m)
                p = p + jnp.max(plsc.all_reduce_population_count(m))
                return i + 1, p, slot, pend, pptr

            def outer_cond(t):
                i = t[0]
                return i < _CHUNK // 16

            def outer_body(t):
                i, p, slot, pend, pptr = lax.while_loop(inner_cond,
                                                        inner_body, t)
                full = p > _FB - 16

                @pl.when(full)
                def _():
                    @pl.when(pend == 1)
                    def _():
                        wait_gather(1 - slot)
                        apply_slot(1 - slot, pptr)
                    start_gather(slot)

                return (i,
                        jnp.where(full, 0, p),
                        jnp.where(full, 1 - slot, slot),
                        jnp.where(full, 1, pend),
                        jnp.where(full, p, pptr))

            return lax.while_loop(outer_cond, outer_body,
                                  (jnp.int32(0),) + st)[1:]

        st = lax.fori_loop(0, _NCHUNK, chunk_body,
                           (jnp.int32(0), jnp.int32(0), jnp.int32(0),
                            jnp.int32(0)))
        ptr, slot, pend, pptr = st

        @pl.when(pend == 1)
        def _():
            wait_gather(1 - slot)
            apply_slot(1 - slot, pptr)

        @pl.when(ptr > 0)
        def _():
            start_gather(slot)
            wait_gather(slot)
            apply_slot(slot, ptr)

        pltpu.sync_copy(agg, out_hbm.at[pl.ds(lo, _ROWS)])

    return body(h, src, dst)


def kernel(x, edge_index, batch, gn1_w, gn1_b, gn1_ms, Wl, bl, Wr,
           gn2_w, gn2_b, gn2_ms):
    h = _tc_gn1(x, batch, gn1_w, gn1_b, gn1_ms)
    agg = _sc_scatter_max(h, edge_index[0], edge_index[1])[:_N]
    return _tc_tail(x, h, agg, Wl, bl, Wr, batch, gn2_w, gn2_b, gn2_ms)


# speedup vs baseline: 1.0026x; 1.0026x over previous
import functools

import jax
import jax.numpy as jnp
from jax import lax
from jax.experimental import pallas as pl
from jax.experimental.pallas import tpu as pltpu
from jax.experimental.pallas import tpu_sc as plsc

_N = 10000
_E = 320000
_D = 128
_G = 8

_NSUB = 32
_ROWS = 320
_NPAD = _NSUB * _ROWS
_CHUNK = 2560
_NCHUNK = _E // _CHUNK
_NPAIR = _CHUNK // 32
_FB = 128
_NEG = float("-inf")


_HI = lax.Precision.HIGHEST


def _graph_norm_body(x, bt, w, b, ms):
    oh = (bt[:, None] == lax.broadcasted_iota(jnp.int32, (1, _G), 1))
    oh = oh.astype(jnp.float32)
    cnt = jnp.maximum(jnp.sum(oh, axis=0), 1.0)
    sums = lax.dot_general(oh, x, (((0,), (0,)), ((), ())), precision=_HI,
                           preferred_element_type=jnp.float32)
    mean = sums / cnt[:, None]
    meanb = jnp.dot(oh, mean, precision=_HI,
                    preferred_element_type=jnp.float32)
    out0 = x - ms * meanb
    var = lax.dot_general(oh, out0 * out0, (((0,), (0,)), ((), ())),
                          precision=_HI,
                          preferred_element_type=jnp.float32) / cnt[:, None]
    istd = lax.rsqrt(var + 1e-5)
    istdb = jnp.dot(oh, istd, precision=_HI,
                    preferred_element_type=jnp.float32)
    return w * (out0 * istdb) + b


def _gn1_kernel(x_ref, bt_ref, w_ref, b_ref, ms_ref, o_ref):
    o_ref[...] = _graph_norm_body(x_ref[...], bt_ref[...], w_ref[...],
                                  b_ref[...], ms_ref[...])


def _tc_gn1(x, batch, w, b, ms):
    return pl.pallas_call(
        _gn1_kernel,
        out_shape=jax.ShapeDtypeStruct((_N, _D), jnp.float32),
    )(x, batch, w, b, ms)


def _tail_kernel(x_ref, h_ref, agg_ref, Wl_ref, bl_ref, Wr_ref, bt_ref,
                 w_ref, b_ref, ms_ref, o_ref):
    agg = agg_ref[...]
    agg = jnp.where(agg == _NEG, 0.0, agg)
    y = (lax.dot_general(agg, Wl_ref[...], (((1,), (1,)), ((), ())),
                         precision=_HI, preferred_element_type=jnp.float32)
         + bl_ref[...]
         + lax.dot_general(h_ref[...], Wr_ref[...], (((1,), (1,)), ((), ())),
                           precision=_HI, preferred_element_type=jnp.float32))
    r = jnp.maximum(x_ref[...] + y, 0.0)
    o_ref[...] = _graph_norm_body(r, bt_ref[...], w_ref[...], b_ref[...],
                                  ms_ref[...])


def _tc_tail(x, h, agg, Wl, bl, Wr, batch, w2, b2, ms2):
    return pl.pallas_call(
        _tail_kernel,
        out_shape=jax.ShapeDtypeStruct((_N, _D), jnp.float32),
    )(x, h, agg, Wl, bl, Wr, batch, w2, b2, ms2)


def _sc_scatter_max(h, src, dst):
    mesh = plsc.VectorSubcoreMesh(core_axis_name="c", subcore_axis_name="s")

    @functools.partial(
        pl.kernel,
        out_type=jax.ShapeDtypeStruct((_NPAD, _D), jnp.float32),
        mesh=mesh,
        compiler_params=pltpu.CompilerParams(needs_layout_passes=False),
        scratch_types=[
            pltpu.VMEM((_ROWS, _D), jnp.float32),
            pltpu.VMEM((_CHUNK,), jnp.int32),
            pltpu.VMEM((_CHUNK,), jnp.int32),
            pltpu.VMEM((_FB,), jnp.int32),
            pltpu.VMEM((_FB,), jnp.int32),
            pltpu.VMEM((_FB + 16,), jnp.int32),
            pltpu.VMEM((_FB + 16,), jnp.int32),
            pltpu.VMEM((_FB, _D), jnp.float32),
            pltpu.VMEM((_FB, _D), jnp.float32),
            pltpu.SemaphoreType.DMA,
            pltpu.SemaphoreType.DMA,
        ],
    )
    def body(h_hbm, src_hbm, dst_hbm, out_hbm, agg, dbuf, sbuf, csrc0, csrc1,
             cdl0, cdl1, rows0, rows1, sem0, sem1):
        wid = lax.axis_index("s") * 2 + lax.axis_index("c")
        lo = wid * _ROWS

        neg = jnp.full((16,), _NEG, jnp.float32)
        zero16 = jnp.zeros((16,), jnp.int32)

        def init_row(r, carry):
            for c in range(_D // 16):
                agg[r, pl.ds(c * 16, 16)] = neg
            return carry
        lax.fori_loop(0, _ROWS, init_row, 0)
        for g in range(_FB // 16):
            csrc0[pl.ds(g * 16, 16)] = zero16
            csrc1[pl.ds(g * 16, 16)] = zero16

        def make_slot(csrc, cdl, rows, sem):
            def start():
                pltpu.async_copy(h_hbm.at[csrc], rows, sem)

            def wait():
                pltpu.make_async_copy(h_hbm.at[csrc], rows, sem).wait()

            def apply(n):
                def apply_j(j, carry):
                    dl = cdl[pl.ds(j, 16)][0]
                    for c in range(_D // 16):
                        slc = pl.ds(c * 16, 16)
                        agg[dl, slc] = jnp.maximum(agg[dl, slc], rows[j, slc])
                    return carry
                lax.fori_loop(0, n, apply_j, 0)

            def scan(i, p):
                def cond(t):
                    i2, p2 = t
                    return (i2 < _CHUNK // 16) & (p2 <= _FB - 16)

                def bdy(t):
                    i2, p2 = t
                    d = dbuf[pl.ds(i2 * 16, 16)]
                    s = sbuf[pl.ds(i2 * 16, 16)]
                    dl = d - lo
                    m = (dl >= 0) & (dl < _ROWS)
                    plsc.store_compressed(csrc.at[pl.ds(p2, 16)], s, mask=m)
                    plsc.store_compressed(cdl.at[pl.ds(p2, 16)], dl, mask=m)
                    p2 = p2 + jnp.max(plsc.all_reduce_population_count(m))
                    return i2 + 1, p2

                return lax.while_loop(cond, bdy, (i, p))

            return start, wait, apply, scan

        start0, wait0, apply0, scan0 = make_slot(csrc0, cdl0, rows0, sem0)
        start1, wait1, apply1, scan1 = make_slot(csrc1, cdl1, rows1, sem1)

        def chunk_body(k, st):
            pltpu.sync_copy(dst_hbm.at[pl.ds(k * _CHUNK, _CHUNK)], dbuf)
            pltpu.sync_copy(src_hbm.at[pl.ds(k * _CHUNK, _CHUNK)], sbuf)

            def outer_cond(t):
                return t[0] < _CHUNK // 16

            def outer_body(t):
                i, p, parity, pend, pptr = t
                i, p = lax.cond(parity == 0,
                                lambda: scan0(i, p),
                                lambda: scan1(i, p))
                full = p > _FB - 16

                @pl.when(full)
                def _():
                    @pl.when(pend == 1)
                    def _():
                        @pl.when(parity == 0)
                        def _():
                            wait1()
                            apply1(pptr)

                        @pl.when(parity == 1)
                        def _():
                            wait0()
                            apply0(pptr)

                    @pl.when(parity == 0)
                    def _():
                        start0()

                    @pl.when(parity == 1)
                    def _():
                        start1()

                return (i,
                        jnp.where(full, 0, p),
                        jnp.where(full, 1 - parity, parity),
                        jnp.where(full, 1, pend),
                        jnp.where(full, p, pptr))

            return lax.while_loop(outer_cond, outer_body,
                                  (jnp.int32(0),) + st)[1:]

        st = lax.fori_loop(0, _NCHUNK, chunk_body,
                           (jnp.int32(0), jnp.int32(0), jnp.int32(0),
                            jnp.int32(0)))
        ptr, parity, pend, pptr = st

        @pl.when(pend == 1)
        def _():
            @pl.when(parity == 0)
            def _():
                wait1()
                apply1(pptr)

            @pl.when(parity == 1)
            def _():
                wait0()
                apply0(pptr)

        @pl.when(ptr > 0)
        def _():
            @pl.when(parity == 0)
            def _():
                start0()
                wait0()
                apply0(ptr)

            @pl.when(parity == 1)
            def _():
                start1()
                wait1()
                apply1(ptr)

        pltpu.sync_copy(agg, out_hbm.at[pl.ds(lo, _ROWS)])

    return body(h, src, dst)


def kernel(x, edge_index, batch, gn1_w, gn1_b, gn1_ms, Wl, bl, Wr,
           gn2_w, gn2_b, gn2_ms):
    h = _tc_gn1(x, batch, gn1_w, gn1_b, gn1_ms)
    agg = _sc_scatter_max(h, edge_index[0], edge_index[1])[:_N]
    return _tc_tail(x, h, agg, Wl, bl, Wr, batch, gn2_w, gn2_b, gn2_ms)


# speedup vs baseline: 1.0107x; 1.0081x over previous
import functools

import jax
import jax.numpy as jnp
from jax import lax
from jax.experimental import pallas as pl
from jax.experimental.pallas import tpu as pltpu
from jax.experimental.pallas import tpu_sc as plsc

_N = 10000
_E = 320000
_D = 128
_G = 8

_NSUB = 32
_ROWS = 320
_NPAD = _NSUB * _ROWS
_CHUNK = 2560
_NCHUNK = _E // _CHUNK
_FB = 128
_NEG = float("-inf")

_HI = lax.Precision.HIGHEST


def _graph_norm_body(x, bt, w, b, ms):
    oh = (bt[:, None] == lax.broadcasted_iota(jnp.int32, (1, _G), 1))
    oh = oh.astype(jnp.float32)
    cnt = jnp.maximum(jnp.sum(oh, axis=0), 1.0)
    sums = lax.dot_general(oh, x, (((0,), (0,)), ((), ())), precision=_HI,
                           preferred_element_type=jnp.float32)
    mean = sums / cnt[:, None]
    meanb = jnp.dot(oh, mean, precision=_HI,
                    preferred_element_type=jnp.float32)
    out0 = x - ms * meanb
    var = lax.dot_general(oh, out0 * out0, (((0,), (0,)), ((), ())),
                          precision=_HI,
                          preferred_element_type=jnp.float32) / cnt[:, None]
    istd = lax.rsqrt(var + 1e-5)
    istdb = jnp.dot(oh, istd, precision=_HI,
                    preferred_element_type=jnp.float32)
    return w * (out0 * istdb) + b


def _gn1_kernel(x_ref, bt_ref, w_ref, b_ref, ms_ref, o_ref):
    o_ref[...] = _graph_norm_body(x_ref[...], bt_ref[...], w_ref[...],
                                  b_ref[...], ms_ref[...])


def _tc_gn1(x, batch, w, b, ms):
    return pl.pallas_call(
        _gn1_kernel,
        out_shape=jax.ShapeDtypeStruct((_N, _D), jnp.float32),
    )(x, batch, w, b, ms)


def _tail_kernel(x_ref, h_ref, agg_ref, Wl_ref, bl_ref, Wr_ref, bt_ref,
                 w_ref, b_ref, ms_ref, o_ref):
    agg = agg_ref[...]
    agg = jnp.where(agg == _NEG, 0.0, agg)
    y = (lax.dot_general(agg, Wl_ref[...], (((1,), (1,)), ((), ())),
                         precision=_HI, preferred_element_type=jnp.float32)
         + bl_ref[...]
         + lax.dot_general(h_ref[...], Wr_ref[...], (((1,), (1,)), ((), ())),
                           precision=_HI, preferred_element_type=jnp.float32))
    r = jnp.maximum(x_ref[...] + y, 0.0)
    o_ref[...] = _graph_norm_body(r, bt_ref[...], w_ref[...], b_ref[...],
                                  ms_ref[...])


def _tc_tail(x, h, agg, Wl, bl, Wr, batch, w2, b2, ms2):
    return pl.pallas_call(
        _tail_kernel,
        out_shape=jax.ShapeDtypeStruct((_N, _D), jnp.float32),
    )(x, h, agg, Wl, bl, Wr, batch, w2, b2, ms2)


def _sc_scatter_max(h, src, dst):
    mesh = plsc.VectorSubcoreMesh(core_axis_name="c", subcore_axis_name="s")

    @functools.partial(
        pl.kernel,
        out_type=jax.ShapeDtypeStruct((_NPAD, _D), jnp.float32),
        mesh=mesh,
        compiler_params=pltpu.CompilerParams(needs_layout_passes=False),
        scratch_types=[
            pltpu.VMEM((_ROWS, _D), jnp.float32),
            pltpu.VMEM((_CHUNK,), jnp.int32),
            pltpu.VMEM((_CHUNK,), jnp.int32),
            pltpu.VMEM((_FB,), jnp.int32),
            pltpu.VMEM((_FB + 16,), jnp.int32),
            pltpu.VMEM((_FB, _D), jnp.float32),
            pltpu.SemaphoreType.DMA,
        ],
    )
    def body(h_hbm, src_hbm, dst_hbm, out_hbm, agg, dbuf, sbuf, csrc, cdl,
             rows, sem):
        wid = lax.axis_index("s") * 2 + lax.axis_index("c")
        lo = wid * _ROWS

        neg = jnp.full((16,), _NEG, jnp.float32)
        zero16 = jnp.zeros((16,), jnp.int32)

        def init_row(r, carry):
            for c in range(_D // 16):
                agg[r, pl.ds(c * 16, 16)] = neg
            return carry
        lax.fori_loop(0, _ROWS, init_row, 0)
        for g in range(_FB // 16):
            csrc[pl.ds(g * 16, 16)] = zero16

        def flush(n):
            pltpu.async_copy(h_hbm.at[csrc], rows, sem).wait()

            def apply(j, carry):
                dl = cdl[pl.ds(j, 16)][0]
                for c in range(_D // 16):
                    slc = pl.ds(c * 16, 16)
                    agg[dl, slc] = jnp.maximum(agg[dl, slc], rows[j, slc])
                return carry
            lax.fori_loop(0, n, apply, 0)

        def chunk_body(k, ptr):
            pltpu.sync_copy(dst_hbm.at[pl.ds(k * _CHUNK, _CHUNK)], dbuf)
            pltpu.sync_copy(src_hbm.at[pl.ds(k * _CHUNK, _CHUNK)], sbuf)

            def inner_cond(st):
                i, p = st
                return (i < _CHUNK // 16) & (p <= _FB - 16)

            def inner_body(st):
                i, p = st
                d = dbuf[pl.ds(i * 16, 16)]
                s = sbuf[pl.ds(i * 16, 16)]
                dl = d - lo
                m = (dl >= 0) & (dl < _ROWS)
                plsc.store_compressed(csrc.at[pl.ds(p, 16)], s, mask=m)
                plsc.store_compressed(cdl.at[pl.ds(p, 16)], dl, mask=m)
                p = p + jnp.max(plsc.all_reduce_population_count(m))
                return i + 1, p

            def outer_cond(st):
                i, _ = st
                return i < _CHUNK // 16

            def outer_body(st):
                i, p = lax.while_loop(inner_cond, inner_body, st)
                full = p > _FB - 16

                @pl.when(full)
                def _():
                    flush(p)
                return i, jnp.where(full, 0, p)

            _, ptr = lax.while_loop(outer_cond, outer_body,
                                    (jnp.int32(0), ptr))
            return ptr

        ptr = lax.fori_loop(0, _NCHUNK, chunk_body, jnp.int32(0))

        @pl.when(ptr > 0)
        def _():
            flush(ptr)

        pltpu.sync_copy(agg, out_hbm.at[pl.ds(lo, _ROWS)])

    return body(h, src, dst)


def kernel(x, edge_index, batch, gn1_w, gn1_b, gn1_ms, Wl, bl, Wr,
           gn2_w, gn2_b, gn2_ms):
    h = _tc_gn1(x, batch, gn1_w, gn1_b, gn1_ms)
    agg = _sc_scatter_max(h, edge_index[0], edge_index[1])[:_N]
    return _tc_tail(x, h, agg, Wl, bl, Wr, batch, gn2_w, gn2_b, gn2_ms)
